# baseline (speedup 1.0000x reference)
import functools
import jax
import jax.numpy as jnp
from jax import lax
from jax.experimental import pallas as pl
from jax.experimental.pallas import tpu as pltpu
from jax.experimental.pallas import tpu_sc as plsc

_H = 8
_D = 16
_HID = 128
_NC = 2
_NS = 16
_NW = _NC * _NS
_NP = 10240


def _shuf(v, idx):
    return v.at[idx].get(mode="promise_in_bounds")


def _node_proj_body(ns_ref, wq_ref, bq_ref, wk_ref, bk_ref, wv_ref, bv_ref,
                    q_ref, k_ref, msg_ref):
    ns = ns_ref[...]
    q_ref[...] = jnp.dot(ns, wq_ref[...], preferred_element_type=jnp.float32) + bq_ref[...]
    k_ref[...] = jnp.dot(ns, wk_ref[...], preferred_element_type=jnp.float32) + bk_ref[...]
    v = jnp.dot(ns, wv_ref[...], preferred_element_type=jnp.float32) + bv_ref[...]
    msg_ref[...] = v * ns


def _node_proj(ns, Wq, bq, Wk, bk, Wv, bv):
    n = ns.shape[0]
    blk = 400
    grid = n // blk
    mat = pl.BlockSpec((_HID, _HID), lambda i: (0, 0))
    vec = pl.BlockSpec((_HID,), lambda i: (0,))
    row = pl.BlockSpec((blk, _HID), lambda i: (i, 0))
    return pl.pallas_call(
        _node_proj_body,
        grid=(grid,),
        in_specs=[row, mat, vec, mat, vec, mat, vec],
        out_specs=[row, row, row],
        out_shape=[jax.ShapeDtypeStruct((n, _HID), jnp.float32)] * 3,
    )(ns, Wq, bq, Wk, bk, Wv, bv)


def _edge_self_body(es_ref, we_ref, be_ref, m_ref, out_ref):
    es = es_ref[...]
    t = jnp.dot(es, we_ref[...], preferred_element_type=jnp.float32) + be_ref[...]
    sp = t * es
    out_ref[...] = jnp.dot(sp, m_ref[...], preferred_element_type=jnp.float32) * 0.25


def _edge_self(es, We, be):
    e = es.shape[0]
    blk = 2000
    grid = e // blk
    m = jnp.kron(jnp.eye(_H, dtype=jnp.float32), jnp.ones((_D, 1), jnp.float32))
    return pl.pallas_call(
        _edge_self_body,
        grid=(grid,),
        in_specs=[
            pl.BlockSpec((blk, _HID), lambda i: (i, 0)),
            pl.BlockSpec((_HID, _HID), lambda i: (0, 0)),
            pl.BlockSpec((_HID,), lambda i: (0,)),
            pl.BlockSpec((_HID, _H), lambda i: (0, 0)),
        ],
        out_specs=pl.BlockSpec((blk, _H), lambda i: (i, 0)),
        out_shape=jax.ShapeDtypeStruct((e, _H), jnp.float32),
    )(es, We, be, m)


def _node_out_body(u_ref, d0_ref, d1_ref, rb_ref, wno_ref, bno_ref, out_ref):
    r = 1.0 / (d0_ref[...] + d1_ref[...] + 1e-12)
    rb = jnp.dot(r, rb_ref[...], preferred_element_type=jnp.float32)
    agg = u_ref[...] * rb
    out_ref[...] = jnp.dot(agg, wno_ref[...], preferred_element_type=jnp.float32) + bno_ref[...]


def _node_out(u, d0, d1, Wno, bno):
    n = u.shape[0]
    blk = 400
    grid = n // blk
    rb = jnp.concatenate(
        [jnp.kron(jnp.eye(_H, dtype=jnp.float32), jnp.ones((1, _D), jnp.float32)),
         jnp.zeros((16 - _H, _HID), jnp.float32)], axis=0)
    row = pl.BlockSpec((blk, _HID), lambda i: (i, 0))
    drow = pl.BlockSpec((blk, 16), lambda i: (i, 0))
    return pl.pallas_call(
        _node_out_body,
        grid=(grid,),
        in_specs=[
            row, drow, drow,
            pl.BlockSpec((16, _HID), lambda i: (0, 0)),
            pl.BlockSpec((_HID, _HID), lambda i: (0, 0)),
            pl.BlockSpec((_HID,), lambda i: (0,)),
        ],
        out_specs=row,
        out_shape=jax.ShapeDtypeStruct((n, _HID), jnp.float32),
    )(u, d0, d1, rb, Wno, bno)


def _edge_out_body(att_ref, weo_ref, beo_ref, out_ref):
    out_ref[...] = jnp.dot(att_ref[...], weo_ref[...], preferred_element_type=jnp.float32) + beo_ref[...]


def _edge_out(att, Weo, beo):
    e = att.shape[0]
    blk = 2000
    grid = e // blk
    return pl.pallas_call(
        _edge_out_body,
        grid=(grid,),
        in_specs=[
            pl.BlockSpec((blk, _H), lambda i: (i, 0)),
            pl.BlockSpec((_H, _HID), lambda i: (0, 0)),
            pl.BlockSpec((_HID,), lambda i: (0,)),
        ],
        out_specs=pl.BlockSpec((blk, _HID), lambda i: (i, 0)),
        out_shape=jax.ShapeDtypeStruct((e, _HID), jnp.float32),
    )(att, Weo, beo)


def _sc_scores(q_t, k_t, src, dst, eself_flat):
    e = src.shape[0]
    ew = e // _NW
    cb = 200
    nch = ew // cb
    mesh = plsc.VectorSubcoreMesh(core_axis_name="c", subcore_axis_name="s")

    @functools.partial(
        pl.kernel, mesh=mesh,
        out_type=(
            jax.ShapeDtypeStruct((e * _H,), jnp.float32),
            jax.ShapeDtypeStruct((_NW, 16), jnp.float32),
        ),
        scratch_types=[
            pltpu.VMEM((cb,), jnp.int32),
            pltpu.VMEM((cb,), jnp.int32),
            pltpu.VMEM((cb, _HID), jnp.float32),
            pltpu.VMEM((cb, _HID), jnp.float32),
            pltpu.VMEM((cb * _H,), jnp.float32),
            pltpu.VMEM((cb * _H,), jnp.float32),
            pltpu.VMEM((16,), jnp.float32),
            pltpu.SemaphoreType.DMA,
            pltpu.SemaphoreType.DMA,
        ],
    )
    def body(q_hbm, k_hbm, src_hbm, dst_hbm, es_hbm,
             sc_hbm, pmax_hbm,
             srcv, dstv, qr, kr, esv, scv, mxv, sem1, sem2):
        wid = lax.axis_index("s") * _NC + lax.axis_index("c")
        iot = lax.iota(jnp.int32, 16)
        ix8 = iot ^ 8
        lo8 = iot < _H

        def chunk(ci, mx):
            base = wid * ew + ci * cb
            pltpu.sync_copy(src_hbm.at[pl.ds(base, cb)], srcv)
            pltpu.sync_copy(dst_hbm.at[pl.ds(base, cb)], dstv)
            pltpu.sync_copy(es_hbm.at[pl.ds(base * _H, cb * _H)], esv)
            c1 = pltpu.async_copy(q_hbm.at[srcv], qr, sem1)
            c2 = pltpu.async_copy(k_hbm.at[dstv], kr, sem2)
            c1.wait()
            c2.wait()

            def pair(j, mx):
                accs = []
                for jj in range(2):
                    row = 2 * j + jj
                    acc = qr[row, pl.ds(0, 16)] * kr[row, pl.ds(0, 16)]
                    for t in range(1, _HID // 16):
                        acc = acc + (qr[row, pl.ds(t * 16, 16)] *
                                     kr[row, pl.ds(t * 16, 16)])
                    acc = acc + _shuf(acc, ix8)
                    accs.append(acc)
                s16 = (jnp.where(lo8, accs[0], accs[1]) * 0.25
                       + esv[pl.ds(j * 16, 16)])
                scv[pl.ds(j * 16, 16)] = s16
                return jnp.maximum(mx, s16)

            mx = lax.fori_loop(0, cb // 2, pair, mx)
            pltpu.sync_copy(scv, sc_hbm.at[pl.ds(base * _H, cb * _H)])
            return mx

        mx = lax.fori_loop(0, nch, chunk, jnp.full((16,), -1e30, jnp.float32))
        mxv[...] = mx
        pltpu.sync_copy(mxv, pmax_hbm.at[wid])

    return body(q_t, k_t, src, dst, eself_flat)


def kernel(node_states, edge_index, edge_states, Wq, bq, Wk, bk, Wv, bv,
           We, be, Wno, bno, Weo, beo):
    n = node_states.shape[0]
    e = edge_states.shape[0]
    src = edge_index[0]
    dst = edge_index[1]

    rows = jnp.arange(_HID)
    cols = (rows % _D) * _H + rows // _D
    perm = jnp.zeros((_HID, _HID), jnp.float32).at[rows, cols].set(1.0)
    q_t, k_t, msg = _node_proj(node_states, Wq @ perm, bq @ perm,
                               Wk @ perm, bk @ perm, Wv, bv)
    eself = _edge_self(edge_states, We, be)

    scores_flat, pmax = _sc_scores(q_t, k_t, src, dst, eself.reshape(-1))
    scores = scores_flat.reshape(e, _H)
    gmax = jnp.max(pmax)
    p = jnp.exp(scores - gmax)
    denom = jax.ops.segment_sum(
        p.reshape(-1),
        (src[:, None] * _H + jnp.arange(_H)[None, :]).reshape(-1),
        num_segments=n * _H).reshape(n, _H)
    u = jax.ops.segment_sum(
        p[:, :, None] * msg[dst].reshape(e, _H, _D), src,
        num_segments=n).reshape(n, _HID)
    att = p / (denom[src] + 1e-12)
    d16 = jnp.concatenate([denom, jnp.zeros((n, 16 - _H), jnp.float32)], axis=1)
    node_update = _node_out(u, d16, jnp.zeros_like(d16), Wno, bno)
    edge_update = _edge_out(att, Weo, beo)
    return (node_update, edge_update, att)
